# baseline (speedup 1.0000x reference)
import jax
import jax.numpy as jnp
from jax.experimental import pallas as pl
from jax.experimental.pallas import tpu as pltpu


def _fused_body(x_ref, we_ref, be_ref, wl_ref, bl_ref, wc_ref, b2_ref, o_ref,
                wce_s, w2_s):
    e, h, s = wc_ref.shape

    @pl.when(pl.program_id(0) == 0)
    def _prep_weights():
        wce_s[:, :h] = we_ref[...]
        wce_s[:, h:] = wl_ref[...]
        w2_s[...] = jnp.transpose(wc_ref[...], (1, 0, 2)).reshape(h, e * s)

    xb = x_ref[...]
    zc = jnp.dot(xb, wce_s[...], preferred_element_type=jnp.float32)
    z = zc[:, :h] + be_ref[...]
    lab = zc[:, h:h + e] + bl_ref[...]
    y = jnp.argmax(lab, axis=-1)[:, None]
    out = jnp.dot(z, w2_s[...], preferred_element_type=jnp.float32) + b2_ref[...]
    tn, es = o_ref.shape
    col_expert = jax.lax.broadcasted_iota(jnp.int32, (tn, es), 1) // s
    o_ref[...] = jnp.where(col_expert == y, out, 0.0)


def kernel(x, W_lab, b_lab, W_enc, b_enc, W_clf, b_clf):
    N, D = x.shape
    E, H, S = W_clf.shape
    ES = E * S
    b2 = b_clf.reshape(1, ES)
    bl = b_lab.reshape(1, E)
    be = b_enc.reshape(1, H)

    TN = 2048
    grid = (N // TN,)

    out = pl.pallas_call(
        _fused_body,
        grid=grid,
        in_specs=[
            pl.BlockSpec((TN, D), lambda i: (i, 0)),
            pl.BlockSpec((D, H), lambda i: (0, 0)),
            pl.BlockSpec((1, H), lambda i: (0, 0)),
            pl.BlockSpec((D, E), lambda i: (0, 0)),
            pl.BlockSpec((1, E), lambda i: (0, 0)),
            pl.BlockSpec((E, H, S), lambda i: (0, 0, 0)),
            pl.BlockSpec((1, ES), lambda i: (0, 0)),
        ],
        out_specs=pl.BlockSpec((TN, ES), lambda i: (i, 0)),
        out_shape=jax.ShapeDtypeStruct((N, ES), x.dtype),
        scratch_shapes=[
            pltpu.VMEM((D, H + E), jnp.float32),
            pltpu.VMEM((H, ES), jnp.float32),
        ],
        compiler_params=pltpu.CompilerParams(
            dimension_semantics=("arbitrary",),
        ),
    )(x, W_enc, be, W_lab, bl, W_clf, b2)
    return out


# speedup vs baseline: 1.0549x; 1.0549x over previous
import jax
import jax.numpy as jnp
from jax.experimental import pallas as pl
from jax.experimental.pallas import tpu as pltpu


def _fused_body(x_ref, we_ref, be_ref, wl_ref, bl_ref, wc_ref, b2_ref, o_ref,
                wce_s, w2_s):
    e, h, s = wc_ref.shape

    @pl.when(pl.program_id(0) == 0)
    def _prep_weights():
        wce_s[:, :h] = we_ref[...]
        wce_s[:, h:] = wl_ref[...]
        w2_s[...] = jnp.transpose(wc_ref[...], (1, 0, 2)).reshape(h, e * s)

    xb = x_ref[...]
    zc = jnp.dot(xb, wce_s[...], preferred_element_type=jnp.float32)
    z = zc[:, :h] + be_ref[...]
    lab = zc[:, h:h + e] + bl_ref[...]
    y = jnp.argmax(lab, axis=-1)[:, None]
    out = jnp.dot(z, w2_s[...], preferred_element_type=jnp.float32) + b2_ref[...]
    tn, es = o_ref.shape
    col_expert = jax.lax.broadcasted_iota(jnp.int32, (tn, es), 1) // s
    o_ref[...] = jnp.where(col_expert == y, out, 0.0)


def kernel(x, W_lab, b_lab, W_enc, b_enc, W_clf, b_clf):
    N, D = x.shape
    E, H, S = W_clf.shape
    ES = E * S
    b2 = b_clf.reshape(1, ES)
    bl = b_lab.reshape(1, E)
    be = b_enc.reshape(1, H)

    TN = 4096
    grid = (N // TN,)

    out = pl.pallas_call(
        _fused_body,
        grid=grid,
        in_specs=[
            pl.BlockSpec((TN, D), lambda i: (i, 0)),
            pl.BlockSpec((D, H), lambda i: (0, 0)),
            pl.BlockSpec((1, H), lambda i: (0, 0)),
            pl.BlockSpec((D, E), lambda i: (0, 0)),
            pl.BlockSpec((1, E), lambda i: (0, 0)),
            pl.BlockSpec((E, H, S), lambda i: (0, 0, 0)),
            pl.BlockSpec((1, ES), lambda i: (0, 0)),
        ],
        out_specs=pl.BlockSpec((TN, ES), lambda i: (i, 0)),
        out_shape=jax.ShapeDtypeStruct((N, ES), x.dtype),
        scratch_shapes=[
            pltpu.VMEM((D, H + E), jnp.float32),
            pltpu.VMEM((H, ES), jnp.float32),
        ],
        compiler_params=pltpu.CompilerParams(
            dimension_semantics=("arbitrary",),
        ),
    )(x, W_enc, be, W_lab, bl, W_clf, b2)
    return out
